# baseline (speedup 1.0000x reference)
import functools

import jax
import jax.numpy as jnp
from jax import lax
from jax.experimental import pallas as pl
from jax.experimental.pallas import tpu as pltpu
from jax.experimental.pallas import tpu_sc as plsc

N = 100
K = 101
B = 16384
CSTRIDE = 136
RSTRIDE = 2 * CSTRIDE


def _table_kernel(w0_ref, w1_ref, ew_ref, o_ref):
    w0 = w0_ref[...]
    w1 = w1_ref[...]
    ew = ew_ref[...]
    m = jnp.maximum(w0, w1)
    lse2 = m + jnp.log(jnp.exp(w0 - m) + jnp.exp(w1 - m))
    l0 = w0 - lse2
    l1 = w1 - lse2
    emax = jnp.max(ew, axis=1, keepdims=True)
    esum = jnp.sum(jnp.exp(ew - emax), axis=1, keepdims=True)
    le = ew - emax - jnp.log(esum)
    r = lax.broadcasted_iota(jnp.int32, (N - 1, N), 0)
    gm1 = lax.broadcasted_iota(jnp.int32, (N - 1, N), 1)
    valid = gm1 <= r
    o_ref[...] = jnp.zeros((N, RSTRIDE), jnp.float32)
    o_ref[0 : N - 1, 1 : K] = jnp.where(valid, l0, 0.0)
    o_ref[0 : N - 1, CSTRIDE + 1 : CSTRIDE + K] = jnp.where(valid, l1, 0.0)
    o_ref[N - 1 : N, 0:K] = le


def _build_table(W, endW):
    return pl.pallas_call(
        _table_kernel,
        out_shape=jax.ShapeDtypeStruct((N, RSTRIDE), jnp.float32),
    )(W[:, :, 0], W[:, :, 1], endW)


def _make_sc_kernel():
    info = plsc.get_sparse_core_info()
    nc, ns = info.num_cores, info.num_subcores
    nw = nc * ns
    bpw = B // nw
    groups = bpw // 64
    mesh = plsc.VectorSubcoreMesh(core_axis_name="c", subcore_axis_name="s")

    @functools.partial(
        pl.kernel,
        mesh=mesh,
        out_type=jax.ShapeDtypeStruct((B,), jnp.float32),
        scratch_types=[
            pltpu.VMEM((N, bpw // 4), jnp.int32),
            pltpu.VMEM((N * RSTRIDE,), jnp.float32),
            pltpu.VMEM((bpw,), jnp.float32),
            pltpu.SemaphoreType.DMA,
            pltpu.SemaphoreType.DMA,
        ],
        compiler_params=pltpu.CompilerParams(needs_layout_passes=False),
    )
    def sc_fn(xt_hbm, tbl_hbm, out_hbm, x_v, tbl_v, out_v, sem_x, sem_t):
        wid = lax.axis_index("s") * nc + lax.axis_index("c")
        base = wid * bpw
        cp_x = pltpu.async_copy(xt_hbm.at[wid], x_v, sem_x)
        cp_t = pltpu.async_copy(tbl_hbm, tbl_v, sem_t)
        cp_x.wait()
        cp_t.wait()
        lanes4 = lax.iota(jnp.int32, 16) * 4

        def cbody(c, carry):
            col = c * 64
            w0 = x_v[0, pl.ds(c * 16, 16)]
            gsub = [(w0 >> (8 * k)) & 1 for k in range(4)]
            accs = [jnp.zeros((16,), jnp.float32) for _ in range(4)]
            for j0 in range(1, N, 4):
                js = list(range(j0, min(j0 + 4, N)))
                ws = [x_v[j, pl.ds(c * 16, 16)] for j in js]
                for k in range(4):
                    xs = [(w >> (8 * k)) & 1 for w in ws]
                    pre = []
                    s = None
                    for xv in xs:
                        s = xv if s is None else s + xv
                        pre.append(s)
                    gs = [gsub[k] + p for p in pre]
                    for t, j in enumerate(js):
                        idx = xs[t] * CSTRIDE + gs[t] + (j - 1) * RSTRIDE
                        accs[k] = accs[k] + plsc.load_gather(tbl_v, [idx])
                    gsub[k] = gs[-1]
            for k in range(4):
                acc = accs[k] + plsc.load_gather(
                    tbl_v, [gsub[k] + (N - 1) * RSTRIDE]
                )
                plsc.store_scatter(out_v, [lanes4 + (col + k)], acc)
            return carry

        lax.fori_loop(0, groups, cbody, 0)
        pltpu.sync_copy(out_v, out_hbm.at[pl.ds(base, bpw)])

    return sc_fn


_SC_KERNEL = None


def kernel(x, W, endW):
    global _SC_KERNEL
    if _SC_KERNEL is None:
        _SC_KERNEL = _make_sc_kernel()
    table = _build_table(W, endW)
    xt = lax.bitcast_convert_type(
        x.T.astype(jnp.int8).reshape(N, B // 4, 4), jnp.int32
    )
    xt = xt.reshape(N, 32, B // 128).transpose(1, 0, 2)
    out = _SC_KERNEL(xt, table.reshape(-1))
    return out[:, None]


# speedup vs baseline: 2.0529x; 2.0529x over previous
import functools

import jax
import jax.numpy as jnp
from jax import lax
from jax.experimental import pallas as pl
from jax.experimental.pallas import tpu as pltpu
from jax.experimental.pallas import tpu_sc as plsc

N = 100
K = 101
B = 16384
CSTRIDE = 136
RSTRIDE = 2 * CSTRIDE
NEG = -1e30


def _table_kernel(w0_ref, w1_ref, ew_ref, o_ref):
    w0 = w0_ref[...]
    w1 = w1_ref[...]
    ew = ew_ref[...]
    m = jnp.maximum(w0, w1)
    lse2 = m + jnp.log(jnp.exp(w0 - m) + jnp.exp(w1 - m))
    l0 = w0 - lse2
    l1 = w1 - lse2
    emax = jnp.max(ew, axis=1, keepdims=True)
    esum = jnp.sum(jnp.exp(ew - emax), axis=1, keepdims=True)
    le = ew - emax - jnp.log(esum)
    r = lax.broadcasted_iota(jnp.int32, (N - 1, N), 0)
    gm1 = lax.broadcasted_iota(jnp.int32, (N - 1, N), 1)
    valid = gm1 <= r
    o_ref[...] = jnp.zeros((N, RSTRIDE), jnp.float32)
    o_ref[0 : N - 1, 1 : K] = jnp.where(valid, l0, 0.0)
    o_ref[0 : N - 1, CSTRIDE + 1 : CSTRIDE + K] = jnp.where(valid, l1, 0.0)
    o_ref[N - 1 : N, 0:K] = le


def _build_table(W, endW):
    return pl.pallas_call(
        _table_kernel,
        out_shape=jax.ShapeDtypeStruct((N, RSTRIDE), jnp.float32),
    )(W[:, :, 0], W[:, :, 1], endW)


def _make_sc_kernel():
    info = plsc.get_sparse_core_info()
    nc, ns = info.num_cores, info.num_subcores
    nw = nc * ns
    bpw = B // nw
    groups = bpw // 16
    mesh = plsc.VectorSubcoreMesh(core_axis_name="c", subcore_axis_name="s")

    @functools.partial(
        pl.kernel,
        mesh=mesh,
        out_type=jax.ShapeDtypeStruct((B,), jnp.float32),
        scratch_types=[
            pltpu.VMEM((N, bpw), jnp.int32),
            pltpu.VMEM((N * RSTRIDE,), jnp.float32),
            pltpu.VMEM((bpw,), jnp.float32),
            pltpu.SemaphoreType.DMA,
            pltpu.SemaphoreType.DMA,
        ],
        compiler_params=pltpu.CompilerParams(needs_layout_passes=False),
    )
    def sc_fn(xt_hbm, tbl_hbm, out_hbm, x_v, tbl_v, out_v, sem_x, sem_t):
        wid = lax.axis_index("s") * nc + lax.axis_index("c")
        base = wid * bpw
        cp_x = pltpu.async_copy(xt_hbm.at[:, pl.ds(base, bpw)], x_v, sem_x)
        cp_t = pltpu.async_copy(tbl_hbm, tbl_v, sem_t)
        cp_x.wait()
        cp_t.wait()

        def cbody(c, carry):
            col = c * 16
            g = x_v[0, pl.ds(col, 16)]
            accs = [jnp.zeros((16,), jnp.float32) for _ in range(4)]
            for j0 in range(1, N, 4):
                js = [j for j in range(j0, min(j0 + 4, N))]
                xs = [x_v[j, pl.ds(col, 16)] for j in js]
                pre = []
                s = None
                for xv in xs:
                    s = xv if s is None else s + xv
                    pre.append(s)
                gs = [g + p for p in pre]
                for k, j in enumerate(js):
                    idx = xs[k] * CSTRIDE + gs[k] + (j - 1) * RSTRIDE
                    accs[k % 4] = accs[k % 4] + plsc.load_gather(
                        tbl_v, [idx]
                    )
                g = gs[-1]
            acc = (accs[0] + accs[1]) + (accs[2] + accs[3])
            acc = acc + plsc.load_gather(tbl_v, [g + (N - 1) * RSTRIDE])
            out_v[pl.ds(col, 16)] = acc
            return carry

        lax.fori_loop(0, groups, cbody, 0)
        pltpu.sync_copy(out_v, out_hbm.at[pl.ds(base, bpw)])

    return sc_fn


_SC_KERNEL = None


def kernel(x, W, endW):
    global _SC_KERNEL
    if _SC_KERNEL is None:
        _SC_KERNEL = _make_sc_kernel()
    table = _build_table(W, endW)
    out = _SC_KERNEL(x.T.astype(jnp.int32), table.reshape(-1))
    return out[:, None]
